# Initial kernel scaffold; baseline (speedup 1.0000x reference)
import jax
import jax.numpy as jnp
from jax import lax
from jax.experimental import pallas as pl
from jax.experimental.pallas import tpu as pltpu
from jax.experimental.pallas import tpu_sc as plsc

N_NODES = 10000
N_EDGES = 320000
D = 128
N_WORKERS = 32
EDGES_PER_W = N_EDGES // N_WORKERS
CHUNK = 128
N_FULL = EDGES_PER_W // CHUNK
REM = EDGES_PER_W - N_FULL * CHUNK
ROWS_PER_TILE = N_NODES // 16
CNT_W = 16


def _make_sc_agg(compute_cnt: bool):
    mesh = plsc.VectorSubcoreMesh(core_axis_name="c", subcore_axis_name="s")
    out_type = [jax.ShapeDtypeStruct((2, N_NODES, D), jnp.float32)]
    if compute_cnt:
        out_type.append(jax.ShapeDtypeStruct((2, N_NODES, CNT_W), jnp.float32))
    scratch = [
        pltpu.VMEM((CHUNK,), jnp.int32),
        pltpu.VMEM((CHUNK,), jnp.int32),
        pltpu.VMEM((REM,), jnp.int32),
        pltpu.VMEM((REM,), jnp.int32),
        pltpu.VMEM((CHUNK, D), jnp.float32),
        pltpu.VMEM((REM, D), jnp.float32),
        pltpu.VMEM((CHUNK, CNT_W), jnp.float32),
        pltpu.VMEM_SHARED((N_NODES, D), jnp.float32),
        pltpu.VMEM_SHARED((N_NODES, CNT_W), jnp.float32),
        pltpu.SemaphoreType.DMA,
    ]

    def body(x_hbm, src_hbm, dst_hbm, z_d, z_c, ones_hbm, agg_out, *rest):
        if compute_cnt:
            cnt_out = rest[0]
            scr = rest[1:]
        else:
            cnt_out = None
            scr = rest
        (idx_s, idx_d, idx_s_r, idx_d_r, rows, rows_r, ones_v, agg_acc,
         cnt_acc, sem) = scr

        cid = lax.axis_index("c")
        sid = lax.axis_index("s")
        wid = cid * 16 + sid

        row0 = pl.multiple_of(sid * ROWS_PER_TILE, 8)
        pltpu.sync_copy(z_d, agg_acc.at[pl.ds(row0, ROWS_PER_TILE)])
        if compute_cnt:
            pltpu.sync_copy(z_c, cnt_acc.at[pl.ds(row0, ROWS_PER_TILE)])
            pltpu.sync_copy(ones_hbm, ones_v)
        plsc.subcore_barrier()

        base = wid * EDGES_PER_W

        def do_chunk(off, isrc, idst, rbuf, n):
            pltpu.sync_copy(src_hbm.at[pl.ds(off, n)], isrc)
            pltpu.sync_copy(dst_hbm.at[pl.ds(off, n)], idst)
            pltpu.async_copy(x_hbm.at[isrc], rbuf, sem).wait()
            pltpu.sync_copy(rbuf, agg_acc.at[idst], add=True)
            if compute_cnt:
                pltpu.sync_copy(ones_v.at[pl.ds(0, n)], cnt_acc.at[idst],
                                add=True)

        def loop_body(i):
            off = pl.multiple_of(base + i * CHUNK, 8)
            do_chunk(off, idx_s, idx_d, rows, CHUNK)

        pl.loop(0, N_FULL)(loop_body)
        do_chunk(pl.multiple_of(base + N_FULL * CHUNK, 8),
                 idx_s_r, idx_d_r, rows_r, REM)

        plsc.subcore_barrier()

        pltpu.sync_copy(agg_acc.at[pl.ds(row0, ROWS_PER_TILE)],
                        agg_out.at[cid, pl.ds(row0, ROWS_PER_TILE)])
        if compute_cnt:
            pltpu.sync_copy(cnt_acc.at[pl.ds(row0, ROWS_PER_TILE)],
                            cnt_out.at[cid, pl.ds(row0, ROWS_PER_TILE)])

    return pl.kernel(body, out_type=out_type, mesh=mesh,
                     scratch_types=scratch)


_sc_agg_cnt = _make_sc_agg(True)
_sc_agg = _make_sc_agg(False)


def _tc_layer_body(h_ref, agg_ref, cnt_ref, wl_ref, bl_ref, wr_ref, o_ref):
    agg = agg_ref[0] + agg_ref[1]
    cnt = cnt_ref[0, :, 0:1] + cnt_ref[1, :, 0:1]
    inv = 1.0 / jnp.clip(cnt, 1.0, None)
    mean = agg * inv
    acc = (jnp.dot(mean, wl_ref[...], preferred_element_type=jnp.float32)
           + jnp.dot(h_ref[...], wr_ref[...], preferred_element_type=jnp.float32)
           + bl_ref[...])
    o_ref[...] = jnp.where(acc >= 0.0, acc, 0.01 * acc)


def _tc_final_body(h_ref, agg_ref, cnt_ref, wl_ref, bl_ref, wr_ref,
                   wo_ref, bo_ref, o_ref):
    agg = agg_ref[0] + agg_ref[1]
    cnt = cnt_ref[0, :, 0:1] + cnt_ref[1, :, 0:1]
    inv = 1.0 / jnp.clip(cnt, 1.0, None)
    mean = agg * inv
    acc = (jnp.dot(mean, wl_ref[...], preferred_element_type=jnp.float32)
           + jnp.dot(h_ref[...], wr_ref[...], preferred_element_type=jnp.float32)
           + bl_ref[...])
    hrelu = jnp.where(acc >= 0.0, acc, 0.01 * acc)
    o_ref[...] = (jnp.dot(hrelu, wo_ref[...],
                          preferred_element_type=jnp.float32) + bo_ref[...])


_BLK = 1000
_GRID = N_NODES // _BLK


def _row_specs():
    return [
        pl.BlockSpec((_BLK, D), lambda i: (i, 0)),
        pl.BlockSpec((2, _BLK, D), lambda i: (0, i, 0)),
        pl.BlockSpec((2, _BLK, CNT_W), lambda i: (0, i, 0)),
        pl.BlockSpec((D, D), lambda i: (0, 0)),
        pl.BlockSpec((1, D), lambda i: (0, 0)),
        pl.BlockSpec((D, D), lambda i: (0, 0)),
    ]


_tc_layer = pl.pallas_call(
    _tc_layer_body,
    grid=(_GRID,),
    in_specs=_row_specs(),
    out_specs=pl.BlockSpec((_BLK, D), lambda i: (i, 0)),
    out_shape=jax.ShapeDtypeStruct((N_NODES, D), jnp.float32),
)

_tc_final = pl.pallas_call(
    _tc_final_body,
    grid=(_GRID,),
    in_specs=_row_specs() + [
        pl.BlockSpec((D, 1), lambda i: (0, 0)),
        pl.BlockSpec((1, 1), lambda i: (0, 0)),
    ],
    out_specs=pl.BlockSpec((_BLK, 1), lambda i: (i, 0)),
    out_shape=jax.ShapeDtypeStruct((N_NODES, 1), jnp.float32),
)


@jax.jit
def kernel(x, edge_index, Wl1, bl1, Wr1, Wl2, bl2, Wr2, Wl3, bl3, Wr3, Wo, bo):
    src = edge_index[0].astype(jnp.int32)
    dst = edge_index[1].astype(jnp.int32)
    z_d = jnp.zeros((ROWS_PER_TILE, D), jnp.float32)
    z_c = jnp.zeros((ROWS_PER_TILE, CNT_W), jnp.float32)
    ones = jnp.ones((CHUNK, CNT_W), jnp.float32)

    bl1r = bl1.reshape(1, D)
    bl2r = bl2.reshape(1, D)
    bl3r = bl3.reshape(1, D)
    bor = bo.reshape(1, 1)

    agg1, cnt = _sc_agg_cnt(x, src, dst, z_d, z_c, ones)
    h1 = _tc_layer(x, agg1, cnt, Wl1, bl1r, Wr1)
    (agg2,) = _sc_agg(h1, src, dst, z_d, z_c, ones)
    h2 = _tc_layer(h1, agg2, cnt, Wl2, bl2r, Wr2)
    (agg3,) = _sc_agg(h2, src, dst, z_d, z_c, ones)
    return _tc_final(h2, agg3, cnt, Wl3, bl3r, Wr3, Wo, bor)


# speedup vs baseline: 3.0659x; 3.0659x over previous
import jax
import jax.numpy as jnp
from jax import lax
from jax.experimental import pallas as pl
from jax.experimental.pallas import tpu as pltpu
from jax.experimental.pallas import tpu_sc as plsc

N_NODES = 10000
N_EDGES = 320000
D = 128
N_WORKERS = 32
CHUNK = 128
CPT = 80
E_PAD = N_WORKERS * CPT * CHUNK
ACC_ROWS = N_NODES + 8
ROW_STEP = 624
ROW_SPAN = 640
CNT_W = 16


def _make_sc_agg():
    mesh = plsc.VectorSubcoreMesh(core_axis_name="c", subcore_axis_name="s")
    out_type = [jax.ShapeDtypeStruct((2, N_NODES, D), jnp.float32)]
    scratch = [
        pltpu.VMEM((CHUNK,), jnp.int32),
        pltpu.VMEM((CHUNK,), jnp.int32),
        pltpu.VMEM((CHUNK,), jnp.int32),
        pltpu.VMEM((CHUNK,), jnp.int32),
        pltpu.VMEM((CHUNK, D), jnp.float32),
        pltpu.VMEM((CHUNK, D), jnp.float32),
        pltpu.VMEM_SHARED((ACC_ROWS, D), jnp.float32),
        pltpu.SemaphoreType.DMA,
        pltpu.SemaphoreType.DMA,
        pltpu.SemaphoreType.DMA,
        pltpu.SemaphoreType.DMA,
    ]

    def body(x_hbm, src_hbm, dst_hbm, z_d, agg_out,
             idx_s0, idx_s1, idx_d0, idx_d1, rows0, rows1, agg_acc,
             sem_g0, sem_g1, sem_s0, sem_s1):
        cid = lax.axis_index("c")
        sid = lax.axis_index("s")
        wid = cid * 16 + sid

        base = wid * CPT * CHUNK

        row0 = pl.multiple_of(sid * ROW_STEP, 8)
        pltpu.sync_copy(z_d, rows0)
        for k in range(ROW_SPAN // CHUNK):
            r = pl.multiple_of(row0 + k * CHUNK, 8)
            pltpu.sync_copy(rows0, agg_acc.at[pl.ds(r, CHUNK)])
        plsc.subcore_barrier()

        def gather_start(c, ibuf, rbuf, sem):
            off = pl.multiple_of(base + c * CHUNK, 8)
            pltpu.sync_copy(src_hbm.at[pl.ds(off, CHUNK)], ibuf)
            pltpu.async_copy(x_hbm.at[ibuf], rbuf, sem)

        def gather_wait(ibuf, rbuf, sem):
            pltpu.make_async_copy(x_hbm.at[ibuf], rbuf, sem).wait()

        def scatter_start(c, ibuf, rbuf, sem):
            off = pl.multiple_of(base + c * CHUNK, 8)
            pltpu.sync_copy(dst_hbm.at[pl.ds(off, CHUNK)], ibuf)
            pltpu.async_copy(rbuf, agg_acc.at[ibuf], sem, add=True)

        def scatter_wait(ibuf, rbuf, sem):
            pltpu.make_async_copy(rbuf, agg_acc.at[ibuf], sem).wait()

        gather_start(0, idx_s0, rows0, sem_g0)
        gather_start(1, idx_s1, rows1, sem_g1)

        def loop_body(i):
            gather_wait(idx_s0, rows0, sem_g0)
            scatter_start(i, idx_d0, rows0, sem_s0)
            gather_wait(idx_s1, rows1, sem_g1)
            scatter_start(i + 1, idx_d1, rows1, sem_s1)
            scatter_wait(idx_d0, rows0, sem_s0)

            @pl.when(i + 2 < CPT)
            def _():
                gather_start(i + 2, idx_s0, rows0, sem_g0)

            scatter_wait(idx_d1, rows1, sem_s1)

            @pl.when(i + 3 < CPT)
            def _():
                gather_start(i + 3, idx_s1, rows1, sem_g1)

        pl.loop(0, CPT, step=2)(loop_body)

        plsc.subcore_barrier()

        for k in range(ROW_SPAN // CHUNK):
            r = pl.multiple_of(row0 + k * CHUNK, 8)
            pltpu.sync_copy(agg_acc.at[pl.ds(r, CHUNK)], rows0)
            pltpu.sync_copy(rows0, agg_out.at[cid, pl.ds(r, CHUNK)])

    return pl.kernel(body, out_type=out_type, mesh=mesh,
                     scratch_types=scratch)


def _make_sc_cnt():
    mesh = plsc.VectorSubcoreMesh(core_axis_name="c", subcore_axis_name="s")
    out_type = [jax.ShapeDtypeStruct((2, N_NODES, D), jnp.float32)]
    scratch = [
        pltpu.VMEM((CHUNK,), jnp.int32),
        pltpu.VMEM((CHUNK, D), jnp.float32),
        pltpu.VMEM_SHARED((ACC_ROWS, D), jnp.float32),
        pltpu.SemaphoreType.DMA,
    ]

    def body(dst_hbm, z_c, ones_hbm, cnt_out, idx_d, ones_v, cnt_acc, sem):
        cid = lax.axis_index("c")
        sid = lax.axis_index("s")
        wid = cid * 16 + sid
        base = wid * CPT * CHUNK

        row0 = pl.multiple_of(sid * ROW_STEP, 8)
        pltpu.sync_copy(z_c, ones_v)
        for k in range(ROW_SPAN // CHUNK):
            r = pl.multiple_of(row0 + k * CHUNK, 8)
            pltpu.sync_copy(ones_v, cnt_acc.at[pl.ds(r, CHUNK)])
        pltpu.sync_copy(ones_hbm, ones_v)
        plsc.subcore_barrier()

        def loop_body(i):
            off = pl.multiple_of(base + i * CHUNK, 8)
            pltpu.sync_copy(dst_hbm.at[pl.ds(off, CHUNK)], idx_d)
            pltpu.sync_copy(ones_v, cnt_acc.at[idx_d], add=True)

        pl.loop(0, CPT)(loop_body)

        plsc.subcore_barrier()

        for k in range(ROW_SPAN // CHUNK):
            r = pl.multiple_of(row0 + k * CHUNK, 8)
            pltpu.sync_copy(cnt_acc.at[pl.ds(r, CHUNK)], ones_v)
            pltpu.sync_copy(ones_v, cnt_out.at[cid, pl.ds(r, CHUNK)])

    return pl.kernel(body, out_type=out_type, mesh=mesh,
                     scratch_types=scratch)


_sc_agg = _make_sc_agg()
_sc_cnt = _make_sc_cnt()


def _tc_layer_body(h_ref, agg_ref, cnt_ref, wl_ref, bl_ref, wr_ref, o_ref):
    agg = agg_ref[0] + agg_ref[1]
    cnt = cnt_ref[0, :, 0:1] + cnt_ref[1, :, 0:1]
    inv = 1.0 / jnp.clip(cnt, 1.0, None)
    mean = agg * inv
    acc = (jnp.dot(mean, wl_ref[...], preferred_element_type=jnp.float32)
           + jnp.dot(h_ref[...], wr_ref[...], preferred_element_type=jnp.float32)
           + bl_ref[...])
    o_ref[...] = jnp.where(acc >= 0.0, acc, 0.01 * acc)


def _tc_final_body(h_ref, agg_ref, cnt_ref, wl_ref, bl_ref, wr_ref,
                   wo_ref, bo_ref, o_ref):
    agg = agg_ref[0] + agg_ref[1]
    cnt = cnt_ref[0, :, 0:1] + cnt_ref[1, :, 0:1]
    inv = 1.0 / jnp.clip(cnt, 1.0, None)
    mean = agg * inv
    acc = (jnp.dot(mean, wl_ref[...], preferred_element_type=jnp.float32)
           + jnp.dot(h_ref[...], wr_ref[...], preferred_element_type=jnp.float32)
           + bl_ref[...])
    hrelu = jnp.where(acc >= 0.0, acc, 0.01 * acc)
    o_ref[...] = (jnp.dot(hrelu, wo_ref[...],
                          preferred_element_type=jnp.float32) + bo_ref[...])


_BLK = 1000
_GRID = N_NODES // _BLK


def _row_specs():
    return [
        pl.BlockSpec((_BLK, D), lambda i: (i, 0)),
        pl.BlockSpec((2, _BLK, D), lambda i: (0, i, 0)),
        pl.BlockSpec((2, _BLK, D), lambda i: (0, i, 0)),
        pl.BlockSpec((D, D), lambda i: (0, 0)),
        pl.BlockSpec((1, D), lambda i: (0, 0)),
        pl.BlockSpec((D, D), lambda i: (0, 0)),
    ]


_tc_layer = pl.pallas_call(
    _tc_layer_body,
    grid=(_GRID,),
    in_specs=_row_specs(),
    out_specs=pl.BlockSpec((_BLK, D), lambda i: (i, 0)),
    out_shape=jax.ShapeDtypeStruct((N_NODES, D), jnp.float32),
)

_tc_final = pl.pallas_call(
    _tc_final_body,
    grid=(_GRID,),
    in_specs=_row_specs() + [
        pl.BlockSpec((D, 1), lambda i: (0, 0)),
        pl.BlockSpec((1, 1), lambda i: (0, 0)),
    ],
    out_specs=pl.BlockSpec((_BLK, 1), lambda i: (i, 0)),
    out_shape=jax.ShapeDtypeStruct((N_NODES, 1), jnp.float32),
)


@jax.jit
def kernel(x, edge_index, Wl1, bl1, Wr1, Wl2, bl2, Wr2, Wl3, bl3, Wr3, Wo, bo):
    src = edge_index[0].astype(jnp.int32)
    dst = edge_index[1].astype(jnp.int32)
    pad = E_PAD - N_EDGES
    srcp = jnp.concatenate([src, jnp.zeros((pad,), jnp.int32)])
    dstp = jnp.concatenate([dst, jnp.full((pad,), N_NODES, jnp.int32)])

    z_d = jnp.zeros((CHUNK, D), jnp.float32)
    ones = jnp.ones((CHUNK, D), jnp.float32)

    bl1r = bl1.reshape(1, D)
    bl2r = bl2.reshape(1, D)
    bl3r = bl3.reshape(1, D)
    bor = bo.reshape(1, 1)

    (cnt,) = _sc_cnt(dstp, z_d, ones)
    (agg1,) = _sc_agg(x, srcp, dstp, z_d)
    h1 = _tc_layer(x, agg1, cnt, Wl1, bl1r, Wr1)
    (agg2,) = _sc_agg(h1, srcp, dstp, z_d)
    h2 = _tc_layer(h1, agg2, cnt, Wl2, bl2r, Wr2)
    (agg3,) = _sc_agg(h2, srcp, dstp, z_d)
    return _tc_final(h2, agg3, cnt, Wl3, bl3r, Wr3, Wo, bor)


# speedup vs baseline: 3.1188x; 1.0173x over previous
import jax
import jax.numpy as jnp
from jax import lax
from jax.experimental import pallas as pl
from jax.experimental.pallas import tpu as pltpu
from jax.experimental.pallas import tpu_sc as plsc

N_NODES = 10000
N_EDGES = 320000
D = 128
N_WORKERS = 32
CHUNK = 128
CPT = 80
E_PAD = N_WORKERS * CPT * CHUNK
ACC_ROWS = N_NODES + 8
ROW_STEP = 624
ROW_SPAN = 640
CNT_W = 16
NBUF = 3


def _make_sc_agg():
    mesh = plsc.VectorSubcoreMesh(core_axis_name="c", subcore_axis_name="s")
    out_type = [jax.ShapeDtypeStruct((2, N_NODES, D), jnp.float32)]
    scratch = (
        [pltpu.VMEM((CHUNK,), jnp.int32) for _ in range(NBUF)]
        + [pltpu.VMEM((CHUNK,), jnp.int32) for _ in range(NBUF)]
        + [pltpu.VMEM((CHUNK, D), jnp.float32) for _ in range(NBUF)]
        + [pltpu.VMEM_SHARED((ACC_ROWS, D), jnp.float32)]
        + [pltpu.SemaphoreType.DMA for _ in range(NBUF)]
        + [pltpu.SemaphoreType.DMA for _ in range(NBUF)]
    )

    def body(x_hbm, src_hbm, dst_hbm, z_d, agg_out, *scr):
        idx_s = scr[0:NBUF]
        idx_d = scr[NBUF:2 * NBUF]
        rows = scr[2 * NBUF:3 * NBUF]
        agg_acc = scr[3 * NBUF]
        sem_g = scr[3 * NBUF + 1:4 * NBUF + 1]
        sem_s = scr[4 * NBUF + 1:5 * NBUF + 1]

        cid = lax.axis_index("c")
        sid = lax.axis_index("s")
        wid = cid * 16 + sid

        base = wid * CPT * CHUNK

        row0 = pl.multiple_of(sid * ROW_STEP, 8)
        pltpu.sync_copy(z_d, rows[0])
        for k in range(ROW_SPAN // CHUNK):
            r = pl.multiple_of(row0 + k * CHUNK, 8)
            pltpu.sync_copy(rows[0], agg_acc.at[pl.ds(r, CHUNK)])
        plsc.subcore_barrier()

        def gather_start(c, b):
            off = pl.multiple_of(base + c * CHUNK, 8)
            pltpu.sync_copy(src_hbm.at[pl.ds(off, CHUNK)], idx_s[b])
            pltpu.async_copy(x_hbm.at[idx_s[b]], rows[b], sem_g[b])

        def gather_wait(b):
            pltpu.make_async_copy(x_hbm.at[idx_s[b]], rows[b],
                                  sem_g[b]).wait()

        def scatter_start(c, b):
            off = pl.multiple_of(base + c * CHUNK, 8)
            pltpu.sync_copy(dst_hbm.at[pl.ds(off, CHUNK)], idx_d[b])
            pltpu.async_copy(rows[b], agg_acc.at[idx_d[b]], sem_s[b],
                             add=True)

        def scatter_wait(b):
            pltpu.make_async_copy(rows[b], agg_acc.at[idx_d[b]],
                                  sem_s[b]).wait()

        for b in range(NBUF):
            gather_start(b, b)

        def loop_body(i):
            for b in range(NBUF):
                @pl.when(i + b < CPT)
                def _(b=b):
                    gather_wait(b)
                    scatter_start(i + b, b)
            for b in range(NBUF):
                @pl.when(i + b < CPT)
                def _(b=b):
                    scatter_wait(b)

                    @pl.when(i + NBUF + b < CPT)
                    def _():
                        gather_start(i + NBUF + b, b)

        pl.loop(0, CPT, step=NBUF)(loop_body)

        plsc.subcore_barrier()

        for k in range(ROW_SPAN // CHUNK):
            r = pl.multiple_of(row0 + k * CHUNK, 8)
            pltpu.sync_copy(agg_acc.at[pl.ds(r, CHUNK)], rows[0])
            pltpu.sync_copy(rows[0], agg_out.at[cid, pl.ds(r, CHUNK)])

    return pl.kernel(body, out_type=out_type, mesh=mesh,
                     scratch_types=scratch)


def _make_sc_cnt():
    mesh = plsc.VectorSubcoreMesh(core_axis_name="c", subcore_axis_name="s")
    out_type = [jax.ShapeDtypeStruct((2, N_NODES, D), jnp.float32)]
    scratch = [
        pltpu.VMEM((CHUNK,), jnp.int32),
        pltpu.VMEM((CHUNK, D), jnp.float32),
        pltpu.VMEM_SHARED((ACC_ROWS, D), jnp.float32),
        pltpu.SemaphoreType.DMA,
    ]

    def body(dst_hbm, z_c, ones_hbm, cnt_out, idx_d, ones_v, cnt_acc, sem):
        cid = lax.axis_index("c")
        sid = lax.axis_index("s")
        wid = cid * 16 + sid
        base = wid * CPT * CHUNK

        row0 = pl.multiple_of(sid * ROW_STEP, 8)
        pltpu.sync_copy(z_c, ones_v)
        for k in range(ROW_SPAN // CHUNK):
            r = pl.multiple_of(row0 + k * CHUNK, 8)
            pltpu.sync_copy(ones_v, cnt_acc.at[pl.ds(r, CHUNK)])
        pltpu.sync_copy(ones_hbm, ones_v)
        plsc.subcore_barrier()

        def loop_body(i):
            off = pl.multiple_of(base + i * CHUNK, 8)
            pltpu.sync_copy(dst_hbm.at[pl.ds(off, CHUNK)], idx_d)
            pltpu.sync_copy(ones_v, cnt_acc.at[idx_d], add=True)

        pl.loop(0, CPT)(loop_body)

        plsc.subcore_barrier()

        for k in range(ROW_SPAN // CHUNK):
            r = pl.multiple_of(row0 + k * CHUNK, 8)
            pltpu.sync_copy(cnt_acc.at[pl.ds(r, CHUNK)], ones_v)
            pltpu.sync_copy(ones_v, cnt_out.at[cid, pl.ds(r, CHUNK)])

    return pl.kernel(body, out_type=out_type, mesh=mesh,
                     scratch_types=scratch)


_sc_agg = _make_sc_agg()
_sc_cnt = _make_sc_cnt()


def _tc_layer_body(h_ref, agg_ref, cnt_ref, wl_ref, bl_ref, wr_ref, o_ref):
    agg = agg_ref[0] + agg_ref[1]
    cnt = cnt_ref[0, :, 0:1] + cnt_ref[1, :, 0:1]
    inv = 1.0 / jnp.clip(cnt, 1.0, None)
    mean = agg * inv
    acc = (jnp.dot(mean, wl_ref[...], preferred_element_type=jnp.float32)
           + jnp.dot(h_ref[...], wr_ref[...], preferred_element_type=jnp.float32)
           + bl_ref[...])
    o_ref[...] = jnp.where(acc >= 0.0, acc, 0.01 * acc)


def _tc_final_body(h_ref, agg_ref, cnt_ref, wl_ref, bl_ref, wr_ref,
                   wo_ref, bo_ref, o_ref):
    agg = agg_ref[0] + agg_ref[1]
    cnt = cnt_ref[0, :, 0:1] + cnt_ref[1, :, 0:1]
    inv = 1.0 / jnp.clip(cnt, 1.0, None)
    mean = agg * inv
    acc = (jnp.dot(mean, wl_ref[...], preferred_element_type=jnp.float32)
           + jnp.dot(h_ref[...], wr_ref[...], preferred_element_type=jnp.float32)
           + bl_ref[...])
    hrelu = jnp.where(acc >= 0.0, acc, 0.01 * acc)
    o_ref[...] = (jnp.dot(hrelu, wo_ref[...],
                          preferred_element_type=jnp.float32) + bo_ref[...])


_BLK = 1000
_GRID = N_NODES // _BLK


def _row_specs():
    return [
        pl.BlockSpec((_BLK, D), lambda i: (i, 0)),
        pl.BlockSpec((2, _BLK, D), lambda i: (0, i, 0)),
        pl.BlockSpec((2, _BLK, D), lambda i: (0, i, 0)),
        pl.BlockSpec((D, D), lambda i: (0, 0)),
        pl.BlockSpec((1, D), lambda i: (0, 0)),
        pl.BlockSpec((D, D), lambda i: (0, 0)),
    ]


_tc_layer = pl.pallas_call(
    _tc_layer_body,
    grid=(_GRID,),
    in_specs=_row_specs(),
    out_specs=pl.BlockSpec((_BLK, D), lambda i: (i, 0)),
    out_shape=jax.ShapeDtypeStruct((N_NODES, D), jnp.float32),
)

_tc_final = pl.pallas_call(
    _tc_final_body,
    grid=(_GRID,),
    in_specs=_row_specs() + [
        pl.BlockSpec((D, 1), lambda i: (0, 0)),
        pl.BlockSpec((1, 1), lambda i: (0, 0)),
    ],
    out_specs=pl.BlockSpec((_BLK, 1), lambda i: (i, 0)),
    out_shape=jax.ShapeDtypeStruct((N_NODES, 1), jnp.float32),
)


@jax.jit
def kernel(x, edge_index, Wl1, bl1, Wr1, Wl2, bl2, Wr2, Wl3, bl3, Wr3, Wo, bo):
    src = edge_index[0].astype(jnp.int32)
    dst = edge_index[1].astype(jnp.int32)
    pad = E_PAD - N_EDGES
    srcp = jnp.concatenate([src, jnp.zeros((pad,), jnp.int32)])
    dstp = jnp.concatenate([dst, jnp.full((pad,), N_NODES, jnp.int32)])

    z_d = jnp.zeros((CHUNK, D), jnp.float32)
    ones = jnp.ones((CHUNK, D), jnp.float32)

    bl1r = bl1.reshape(1, D)
    bl2r = bl2.reshape(1, D)
    bl3r = bl3.reshape(1, D)
    bor = bo.reshape(1, 1)

    (cnt,) = _sc_cnt(dstp, z_d, ones)
    (agg1,) = _sc_agg(x, srcp, dstp, z_d)
    h1 = _tc_layer(x, agg1, cnt, Wl1, bl1r, Wr1)
    (agg2,) = _sc_agg(h1, srcp, dstp, z_d)
    h2 = _tc_layer(h1, agg2, cnt, Wl2, bl2r, Wr2)
    (agg3,) = _sc_agg(h2, srcp, dstp, z_d)
    return _tc_final(h2, agg3, cnt, Wl3, bl3r, Wr3, Wo, bor)
